# baseline (speedup 1.0000x reference)
import jax
import jax.numpy as jnp
from jax.experimental import pallas as pl
from jax.experimental.pallas import tpu as pltpu

_ROWS = 32768
_COLS = 2048
_BLOCK_ROWS = 512
_CORES = 2
_INNER = _ROWS // (_BLOCK_ROWS * _CORES)


def _loss_kernel(x_ref, y_ref, out_ref, acc_ref):
    j = pl.program_id(1)

    @pl.when(j == 0)
    def _init():
        acc_ref[0] = 0.0
        acc_ref[1] = 0.0

    x = x_ref[...]
    y = y_ref[...]
    d = x - y
    acc_ref[0] += jnp.sum(d * d)
    acc_ref[1] += jnp.sum(y * y)

    @pl.when(j == _INNER - 1)
    def _finish():
        out_ref[0, 0, 0] = acc_ref[0]
        out_ref[0, 0, 1] = acc_ref[1]


def kernel(x, y):
    x2 = x.reshape(_ROWS, _COLS)
    y2 = y.reshape(_ROWS, _COLS)
    partials = pl.pallas_call(
        _loss_kernel,
        grid=(_CORES, _INNER),
        in_specs=[
            pl.BlockSpec((_BLOCK_ROWS, _COLS), lambda i, j: (i * _INNER + j, 0)),
            pl.BlockSpec((_BLOCK_ROWS, _COLS), lambda i, j: (i * _INNER + j, 0)),
        ],
        out_specs=pl.BlockSpec((1, 1, 2), lambda i, j: (i, 0, 0),
                               memory_space=pltpu.SMEM),
        out_shape=jax.ShapeDtypeStruct((_CORES, 1, 2), jnp.float32),
        scratch_shapes=[pltpu.SMEM((2,), jnp.float32)],
        compiler_params=pltpu.CompilerParams(
            dimension_semantics=("parallel", "arbitrary"),
        ),
    )(x2, y2)
    sums = jnp.sum(partials, axis=(0, 1))
    return sums[0] / (2.0 * sums[1])


# speedup vs baseline: 1.0472x; 1.0472x over previous
import jax
import jax.numpy as jnp
from jax.experimental import pallas as pl
from jax.experimental.pallas import tpu as pltpu

_ROWS = 32768
_COLS = 2048
_BLOCK_ROWS = 1024
_GRID = _ROWS // _BLOCK_ROWS


def _loss_kernel(x_ref, y_ref, out_ref, acc_ref):
    i = pl.program_id(0)

    @pl.when(i == 0)
    def _init():
        acc_ref[0] = 0.0
        acc_ref[1] = 0.0

    x = x_ref[...]
    y = y_ref[...]
    d = x - y
    acc_ref[0] += jnp.sum(d * d)
    acc_ref[1] += jnp.sum(y * y)

    @pl.when(i == _GRID - 1)
    def _finish():
        out_ref[0] = acc_ref[0] / (2.0 * acc_ref[1])


def kernel(x, y):
    x2 = x.reshape(_ROWS, _COLS)
    y2 = y.reshape(_ROWS, _COLS)
    out = pl.pallas_call(
        _loss_kernel,
        grid=(_GRID,),
        in_specs=[
            pl.BlockSpec((_BLOCK_ROWS, _COLS), lambda i: (i, 0)),
            pl.BlockSpec((_BLOCK_ROWS, _COLS), lambda i: (i, 0)),
        ],
        out_specs=pl.BlockSpec(memory_space=pltpu.SMEM),
        out_shape=jax.ShapeDtypeStruct((1,), jnp.float32),
        scratch_shapes=[pltpu.SMEM((2,), jnp.float32)],
    )(x2, y2)
    return out[0]
